# Initial kernel scaffold; baseline (speedup 1.0000x reference)
import jax
import jax.numpy as jnp
from jax.experimental import pallas as pl


def kernel(x, edge_index, edge_attr, batch, lig_w1, lig_b1, lig_w2, lig_b2, prot_w1, prot_b1, prot_w2, prot_b2, conv_w1, conv_b1, conv_w2, conv_b2, edge_w, edge_b, gn_w, gn_b, gn_ms, out_w1, out_b1, out_w2, out_b2):
    raise NotImplementedError("write your pallas kernel here")


# speedup vs baseline: 2.9652x; 2.9652x over previous
import functools

import jax
import jax.numpy as jnp
from jax import lax
from jax.experimental import pallas as pl
from jax.experimental.pallas import tpu as pltpu
from jax.experimental.pallas import tpu_sc as plsc

ATOM = 21


def _embed_body(x_ref, lw1, lb1, lw2, lb2, pw1, pb1, pw2, pb2, h_ref):
    x = x_ref[...]
    n, din = x.shape
    lane = lax.broadcasted_iota(jnp.int32, (n, din), 1)
    res = jnp.sum(jnp.abs(x) * (lane >= ATOM).astype(jnp.float32), axis=1,
                  keepdims=True) > 1e-6
    h_lig = jnp.maximum(
        jnp.maximum(x @ lw1[...] + lb1[...], 0.0) @ lw2[...] + lb2[...], 0.0)
    h_prot = jnp.maximum(
        jnp.maximum(x @ pw1[...] + pb1[...], 0.0) @ pw2[...] + pb2[...], 0.0)
    h_ref[...] = jnp.where(res, h_prot, h_lig)


def _embed(x, lw1, lb1, lw2, lb2, pw1, pb1, pw2, pb2):
    n = x.shape[0]
    return pl.pallas_call(
        _embed_body,
        out_shape=jax.ShapeDtypeStruct((n, lw1.shape[1]), jnp.float32),
    )(x, lw1, lb1, lw2, lb2, pw1, pb1, pw2, pb2)


def _edge_body(ea_ref, w_ref, b_ref, e_ref):
    ea = ea_ref[...]
    nl = w_ref.shape[0]
    for l in range(nl):
        e_ref[l] = ea @ w_ref[l] + b_ref[l]


def _edge_transform(edge_attr, edge_w, edge_b, block_e):
    e, de = edge_attr.shape
    nl, _, h = edge_w.shape
    grid = e // block_e
    return pl.pallas_call(
        _edge_body,
        grid=(grid,),
        in_specs=[
            pl.BlockSpec((block_e, de), lambda i: (i, 0)),
            pl.BlockSpec((nl, de, h), lambda i: (0, 0, 0)),
            pl.BlockSpec((nl, 1, h), lambda i: (0, 0, 0)),
        ],
        out_specs=pl.BlockSpec((nl, block_e, h), lambda i: (0, i, 0)),
        out_shape=jax.ShapeDtypeStruct((nl, e, h), jnp.float32),
    )(edge_attr, edge_w, edge_b.reshape(nl, 1, h))


def _make_sc_agg(n, e_total, h, layer, n_layers):
    NC, NS = 2, 16
    NW = NC * NS
    K = 80
    ep = e_total // NW
    assert ep * NW == e_total and ep % K == 0
    n_chunks = ep // K
    rows_per_sub = (n // NS) // 8 * 8
    tail = n - rows_per_sub * NS
    assert tail % 8 == 0

    mesh = plsc.VectorSubcoreMesh(core_axis_name="c", subcore_axis_name="s",
                                  num_cores=NC, num_subcores=NS)

    def body(h_hbm, e_hbm, src_hbm, dst_hbm, zeros_hbm, out_hbm,
             agg_sh, src0, src1, dst0, dst1, e0, e1, r0, r1,
             se0, se1, sg0, sg1):
        c = lax.axis_index("c")
        s = lax.axis_index("s")
        wid = c * NS + s
        pltpu.sync_copy(zeros_hbm.at[pl.ds(s * rows_per_sub, rows_per_sub)],
                        agg_sh.at[pl.ds(s * rows_per_sub, rows_per_sub)])
        if tail:
            @pl.when(s == NS - 1)
            def _():
                pltpu.sync_copy(zeros_hbm.at[pl.ds(NS * rows_per_sub, tail)],
                                agg_sh.at[pl.ds(NS * rows_per_sub, tail)])
        plsc.subcore_barrier()

        base0 = layer * e_total + wid * ep

        srcs = (src0, src1)
        dsts = (dst0, dst1)
        ebufs = (e0, e1)
        rbufs = (r0, r1)
        esems = (se0, se1)
        gsems = (sg0, sg1)

        def fire(chunk, slot):
            base = base0 + chunk * K
            ib = wid * ep + chunk * K
            pltpu.sync_copy(src_hbm.at[pl.ds(ib, K)], srcs[slot])
            pltpu.sync_copy(dst_hbm.at[pl.ds(ib, K)], dsts[slot])
            ce = pltpu.async_copy(e_hbm.at[pl.ds(base, K)], ebufs[slot],
                                  esems[slot])
            cg = pltpu.async_copy(h_hbm.at[srcs[slot]], rbufs[slot],
                                  gsems[slot])
            return ce, cg

        def consume(chunk, slot):
            pltpu.make_async_copy(e_hbm.at[pl.ds(base0, K)], ebufs[slot],
                                  esems[slot]).wait()
            pltpu.make_async_copy(h_hbm.at[srcs[slot]], rbufs[slot],
                                  gsems[slot]).wait()
            eb = ebufs[slot]
            rb = rbufs[slot]
            for t in range(K):
                for j in range(h // 16):
                    sl = pl.ds(j * 16, 16)
                    rb[t, sl] = jnp.maximum(rb[t, sl] + eb[t, sl], 0.0)
            pltpu.sync_copy(rb, agg_sh.at[dsts[slot]], add=True)

        fire(0, 0)

        def step(i, _):
            fire(2 * i + 1, 1)
            consume(2 * i, 0)

            @pl.when(2 * i + 2 < n_chunks)
            def _():
                fire(2 * i + 2, 0)

            consume(2 * i + 1, 1)
            return 0

        lax.fori_loop(0, n_chunks // 2, step, 0)
        if n_chunks % 2:
            consume(n_chunks - 1, 0)
        plsc.subcore_barrier()
        pltpu.sync_copy(agg_sh.at[pl.ds(s * rows_per_sub, rows_per_sub)],
                        out_hbm.at[c, pl.ds(s * rows_per_sub, rows_per_sub)])
        if tail:
            @pl.when(s == NS - 1)
            def _():
                pltpu.sync_copy(agg_sh.at[pl.ds(NS * rows_per_sub, tail)],
                                out_hbm.at[c, pl.ds(NS * rows_per_sub, tail)])

    return pl.kernel(
        body,
        out_type=jax.ShapeDtypeStruct((NC, n, h), jnp.float32),
        mesh=mesh,
        compiler_params=pltpu.CompilerParams(use_tc_tiling_on_sc=False),
        scratch_types=[
            pltpu.VMEM_SHARED((n, h), jnp.float32),
            pltpu.VMEM((K,), jnp.int32),
            pltpu.VMEM((K,), jnp.int32),
            pltpu.VMEM((K,), jnp.int32),
            pltpu.VMEM((K,), jnp.int32),
            pltpu.VMEM((K, h), jnp.float32),
            pltpu.VMEM((K, h), jnp.float32),
            pltpu.VMEM((K, h), jnp.float32),
            pltpu.VMEM((K, h), jnp.float32),
            pltpu.SemaphoreType.DMA,
            pltpu.SemaphoreType.DMA,
            pltpu.SemaphoreType.DMA,
            pltpu.SemaphoreType.DMA,
        ],
    )


def _update_body(g, h_ref, a_ref, batch_ref, w1, b1, w2, b2, gw, gb, gms,
                 out_ref):
    n, h = h_ref.shape
    z = h_ref[...] + a_ref[0] + a_ref[1]
    hp = jnp.maximum(z @ w1[...] + b1[...], 0.0) @ w2[...] + b2[...]
    b = batch_ref[...]
    oh = (b[:, None] == lax.broadcasted_iota(jnp.int32, (n, g), 1)
          ).astype(jnp.float32)
    oh_t = (b[None, :] == lax.broadcasted_iota(jnp.int32, (g, n), 0)
            ).astype(jnp.float32)
    cnt = jnp.maximum(jnp.sum(oh_t, axis=1, keepdims=True), 1.0)
    mean = (oh_t @ hp) / cnt
    ms = mean * gms[...]
    o = hp - oh @ ms
    var = (oh_t @ (o * o)) / cnt
    rstd = lax.rsqrt(var + 1e-5)
    out_ref[...] = jnp.maximum(gw[...] * o * (oh @ rstd) + gb[...], 0.0)


def _update(h, agg2, batch, w1, b1, w2, b2, gwv, gbv, gmsv):
    n, hh = h.shape
    g = 64
    return pl.pallas_call(
        functools.partial(_update_body, g),
        out_shape=jax.ShapeDtypeStruct((n, hh), jnp.float32),
    )(h, agg2, batch, w1, b1.reshape(1, -1), w2, b2.reshape(1, -1),
      gwv.reshape(1, -1), gbv.reshape(1, -1), gmsv.reshape(1, -1))


def _final_body(h_ref, batch_ref, w1, b1, w2, b2, out_ref):
    n, hh = h_ref.shape
    g = out_ref.shape[0]
    b = batch_ref[...]
    oh_t = (b[None, :] == lax.broadcasted_iota(jnp.int32, (g, n), 0)
            ).astype(jnp.float32)
    pooled = oh_t @ h_ref[...]
    out_ref[...] = jnp.maximum(pooled @ w1[...] + b1[...], 0.0) @ w2[...] \
        + b2[...]


def _final(h, batch, w1, b1, w2, b2, g):
    return pl.pallas_call(
        _final_body,
        out_shape=jax.ShapeDtypeStruct((g, 1), jnp.float32),
    )(h, batch, w1, b1.reshape(1, -1), w2, b2.reshape(1, -1))


def kernel(x, edge_index, edge_attr, batch, lig_w1, lig_b1, lig_w2, lig_b2,
           prot_w1, prot_b1, prot_w2, prot_b2, conv_w1, conv_b1, conv_w2,
           conv_b2, edge_w, edge_b, gn_w, gn_b, gn_ms, out_w1, out_b1,
           out_w2, out_b2):
    n = x.shape[0]
    e_total = edge_index.shape[1]
    nl, _, h = edge_w.shape
    g = 64

    h_nodes = _embed(x, lig_w1, lig_b1.reshape(1, -1), lig_w2,
                     lig_b2.reshape(1, -1), prot_w1, prot_b1.reshape(1, -1),
                     prot_w2, prot_b2.reshape(1, -1))

    e_all = _edge_transform(edge_attr, edge_w, edge_b, block_e=5000)
    e_flat = e_all.reshape(nl * e_total, h)

    src = edge_index[0]
    dst = edge_index[1]
    zeros = jnp.zeros((n, h), jnp.float32)

    for l in range(nl):
        sc = _make_sc_agg(n, e_total, h, l, nl)
        agg2 = sc(h_nodes, e_flat, src, dst, zeros)
        h_nodes = _update(h_nodes, agg2, batch, conv_w1[l], conv_b1[l],
                          conv_w2[l], conv_b2[l], gn_w[l], gn_b[l], gn_ms[l])

    out = _final(h_nodes, batch, out_w1, out_b1, out_w2, out_b2, g)
    return out.reshape(-1)


# speedup vs baseline: 2.9667x; 1.0005x over previous
import functools

import jax
import jax.numpy as jnp
from jax import lax
from jax.experimental import pallas as pl
from jax.experimental.pallas import tpu as pltpu
from jax.experimental.pallas import tpu_sc as plsc

ATOM = 21


def _embed_body(x_ref, lw1, lb1, lw2, lb2, pw1, pb1, pw2, pb2, h_ref):
    x = x_ref[...]
    n, din = x.shape
    lane = lax.broadcasted_iota(jnp.int32, (n, din), 1)
    res = jnp.sum(jnp.abs(x) * (lane >= ATOM).astype(jnp.float32), axis=1,
                  keepdims=True) > 1e-6
    h_lig = jnp.maximum(
        jnp.maximum(x @ lw1[...] + lb1[...], 0.0) @ lw2[...] + lb2[...], 0.0)
    h_prot = jnp.maximum(
        jnp.maximum(x @ pw1[...] + pb1[...], 0.0) @ pw2[...] + pb2[...], 0.0)
    h_ref[...] = jnp.where(res, h_prot, h_lig)


def _embed(x, lw1, lb1, lw2, lb2, pw1, pb1, pw2, pb2):
    n = x.shape[0]
    return pl.pallas_call(
        _embed_body,
        out_shape=jax.ShapeDtypeStruct((n, lw1.shape[1]), jnp.float32),
    )(x, lw1, lb1, lw2, lb2, pw1, pb1, pw2, pb2)


def _edge_body(ea_ref, w_ref, b_ref, e_ref):
    ea = ea_ref[...]
    nl = w_ref.shape[0]
    for l in range(nl):
        e_ref[l] = ea @ w_ref[l] + b_ref[l]


def _edge_transform(edge_attr, edge_w, edge_b, block_e):
    e, de = edge_attr.shape
    nl, _, h = edge_w.shape
    grid = e // block_e
    return pl.pallas_call(
        _edge_body,
        grid=(grid,),
        in_specs=[
            pl.BlockSpec((block_e, de), lambda i: (i, 0)),
            pl.BlockSpec((nl, de, h), lambda i: (0, 0, 0)),
            pl.BlockSpec((nl, 1, h), lambda i: (0, 0, 0)),
        ],
        out_specs=pl.BlockSpec((nl, block_e, h), lambda i: (0, i, 0)),
        out_shape=jax.ShapeDtypeStruct((nl, e, h), jnp.float32),
    )(edge_attr, edge_w, edge_b.reshape(nl, 1, h))


def _make_sc_agg(n, e_total, h, layer, n_layers):
    NC, NS = 2, 16
    NW = NC * NS
    K = 80
    ep = e_total // NW
    assert ep * NW == e_total and ep % K == 0
    n_chunks = ep // K
    rows_per_sub = (n // NS) // 8 * 8
    tail = n - rows_per_sub * NS
    assert tail % 8 == 0

    mesh = plsc.VectorSubcoreMesh(core_axis_name="c", subcore_axis_name="s",
                                  num_cores=NC, num_subcores=NS)

    def body(h_hbm, e_hbm, src_hbm, dst_hbm, zeros_hbm, out_hbm,
             agg_sh, src0, src1, dst0, dst1, e0, e1, r0, r1,
             se0, se1, sg0, sg1):
        c = lax.axis_index("c")
        s = lax.axis_index("s")
        wid = c * NS + s
        pltpu.sync_copy(zeros_hbm.at[pl.ds(s * rows_per_sub, rows_per_sub)],
                        agg_sh.at[pl.ds(s * rows_per_sub, rows_per_sub)])
        if tail:
            @pl.when(s == NS - 1)
            def _():
                pltpu.sync_copy(zeros_hbm.at[pl.ds(NS * rows_per_sub, tail)],
                                agg_sh.at[pl.ds(NS * rows_per_sub, tail)])
        plsc.subcore_barrier()

        base0 = layer * e_total + wid * ep

        srcs = (src0, src1)
        dsts = (dst0, dst1)
        ebufs = (e0, e1)
        rbufs = (r0, r1)
        esems = (se0, se1)
        gsems = (sg0, sg1)

        def fire(chunk, slot):
            base = base0 + chunk * K
            ib = wid * ep + chunk * K
            pltpu.sync_copy(src_hbm.at[pl.ds(ib, K)], srcs[slot])
            pltpu.sync_copy(dst_hbm.at[pl.ds(ib, K)], dsts[slot])
            pltpu.async_copy(e_hbm.at[pl.ds(base, K)], ebufs[slot],
                             esems[slot])
            pltpu.async_copy(h_hbm.at[srcs[slot]], rbufs[slot],
                             gsems[slot])

        def consume(chunk, slot):
            pltpu.make_async_copy(e_hbm.at[pl.ds(base0, K)], ebufs[slot],
                                  esems[slot]).wait()
            pltpu.make_async_copy(h_hbm.at[srcs[slot]], rbufs[slot],
                                  gsems[slot]).wait()
            eb = ebufs[slot]
            rb = rbufs[slot]
            for t in range(K):
                for j in range(h // 16):
                    sl = pl.ds(j * 16, 16)
                    rb[t, sl] = jnp.maximum(rb[t, sl] + eb[t, sl], 0.0)
            pltpu.sync_copy(rb, agg_sh.at[dsts[slot]], add=True)

        fire(0, 0)

        def step(i, _):
            fire(2 * i + 1, 1)
            consume(2 * i, 0)

            @pl.when(2 * i + 2 < n_chunks)
            def _():
                fire(2 * i + 2, 0)

            consume(2 * i + 1, 1)
            return 0

        lax.fori_loop(0, n_chunks // 2, step, 0)
        if n_chunks % 2:
            consume(n_chunks - 1, 0)
        plsc.subcore_barrier()
        pltpu.sync_copy(agg_sh.at[pl.ds(s * rows_per_sub, rows_per_sub)],
                        out_hbm.at[c, pl.ds(s * rows_per_sub, rows_per_sub)])
        if tail:
            @pl.when(s == NS - 1)
            def _():
                pltpu.sync_copy(agg_sh.at[pl.ds(NS * rows_per_sub, tail)],
                                out_hbm.at[c, pl.ds(NS * rows_per_sub, tail)])

    return pl.kernel(
        body,
        out_type=jax.ShapeDtypeStruct((NC, n, h), jnp.float32),
        mesh=mesh,
        compiler_params=pltpu.CompilerParams(use_tc_tiling_on_sc=False),
        scratch_types=[
            pltpu.VMEM_SHARED((n, h), jnp.float32),
            pltpu.VMEM((K,), jnp.int32),
            pltpu.VMEM((K,), jnp.int32),
            pltpu.VMEM((K,), jnp.int32),
            pltpu.VMEM((K,), jnp.int32),
            pltpu.VMEM((K, h), jnp.float32),
            pltpu.VMEM((K, h), jnp.float32),
            pltpu.VMEM((K, h), jnp.float32),
            pltpu.VMEM((K, h), jnp.float32),
            pltpu.SemaphoreType.DMA,
            pltpu.SemaphoreType.DMA,
            pltpu.SemaphoreType.DMA,
            pltpu.SemaphoreType.DMA,
        ],
    )


def _update_body(g, h_ref, a_ref, batch_ref, w1, b1, w2, b2, gw, gb, gms,
                 out_ref):
    n, h = h_ref.shape
    z = h_ref[...] + a_ref[0] + a_ref[1]
    hp = jnp.maximum(z @ w1[...] + b1[...], 0.0) @ w2[...] + b2[...]
    b = batch_ref[...]
    oh = (b[:, None] == lax.broadcasted_iota(jnp.int32, (n, g), 1)
          ).astype(jnp.float32)
    oh_t = (b[None, :] == lax.broadcasted_iota(jnp.int32, (g, n), 0)
            ).astype(jnp.float32)
    cnt = jnp.maximum(jnp.sum(oh_t, axis=1, keepdims=True), 1.0)
    mean = (oh_t @ hp) / cnt
    ms = mean * gms[...]
    o = hp - oh @ ms
    var = (oh_t @ (o * o)) / cnt
    rstd = lax.rsqrt(var + 1e-5)
    out_ref[...] = jnp.maximum(gw[...] * o * (oh @ rstd) + gb[...], 0.0)


def _update(h, agg2, batch, w1, b1, w2, b2, gwv, gbv, gmsv):
    n, hh = h.shape
    g = 64
    return pl.pallas_call(
        functools.partial(_update_body, g),
        out_shape=jax.ShapeDtypeStruct((n, hh), jnp.float32),
    )(h, agg2, batch, w1, b1.reshape(1, -1), w2, b2.reshape(1, -1),
      gwv.reshape(1, -1), gbv.reshape(1, -1), gmsv.reshape(1, -1))


def _final_body(h_ref, batch_ref, w1, b1, w2, b2, out_ref):
    n, hh = h_ref.shape
    g = out_ref.shape[0]
    b = batch_ref[...]
    oh_t = (b[None, :] == lax.broadcasted_iota(jnp.int32, (g, n), 0)
            ).astype(jnp.float32)
    pooled = oh_t @ h_ref[...]
    out_ref[...] = jnp.maximum(pooled @ w1[...] + b1[...], 0.0) @ w2[...] \
        + b2[...]


def _final(h, batch, w1, b1, w2, b2, g):
    return pl.pallas_call(
        _final_body,
        out_shape=jax.ShapeDtypeStruct((g, 1), jnp.float32),
    )(h, batch, w1, b1.reshape(1, -1), w2, b2.reshape(1, -1))


def kernel(x, edge_index, edge_attr, batch, lig_w1, lig_b1, lig_w2, lig_b2,
           prot_w1, prot_b1, prot_w2, prot_b2, conv_w1, conv_b1, conv_w2,
           conv_b2, edge_w, edge_b, gn_w, gn_b, gn_ms, out_w1, out_b1,
           out_w2, out_b2):
    n = x.shape[0]
    e_total = edge_index.shape[1]
    nl, _, h = edge_w.shape
    g = 64

    h_nodes = _embed(x, lig_w1, lig_b1.reshape(1, -1), lig_w2,
                     lig_b2.reshape(1, -1), prot_w1, prot_b1.reshape(1, -1),
                     prot_w2, prot_b2.reshape(1, -1))

    e_all = _edge_transform(edge_attr, edge_w, edge_b, block_e=5000)
    e_flat = e_all.reshape(nl * e_total, h)

    src = edge_index[0]
    dst = edge_index[1]
    zeros = jnp.zeros((n, h), jnp.float32)

    for l in range(nl):
        sc = _make_sc_agg(n, e_total, h, l, nl)
        agg2 = sc(h_nodes, e_flat, src, dst, zeros)
        h_nodes = _update(h_nodes, agg2, batch, conv_w1[l], conv_b1[l],
                          conv_w2[l], conv_b2[l], gn_w[l], gn_b[l], gn_ms[l])

    out = _final(h_nodes, batch, out_w1, out_b1, out_w2, out_b2, g)
    return out.reshape(-1)
